# Initial kernel scaffold; baseline (speedup 1.0000x reference)
import jax
import jax.numpy as jnp
from jax.experimental import pallas as pl


def kernel(x, adj, attrs, mlp_w, mlp_b, ln_g, ln_b, gc1_w, gc1_b, gc2_w, gc2_b, gc3_w, gc3_b):
    raise NotImplementedError("write your pallas kernel here")


# speedup vs baseline: 1.0148x; 1.0148x over previous
import functools

import jax
import jax.numpy as jnp
from jax.experimental import pallas as pl
from jax.experimental.pallas import tpu as pltpu

_N = 10000
_H = 128
_K = 10
_BR = 200
_BP = 1000


def _prologue_body(x_ref, mlp_w_ref, mlp_b_ref, ln_g_ref, ln_b_ref,
                   gc1_w_ref, h0_ref, s1_ref):
    h = jnp.maximum(x_ref[...] @ mlp_w_ref[...].T + mlp_b_ref[...], 0.0)
    mu = jnp.mean(h, axis=-1, keepdims=True)
    var = jnp.mean((h - mu) ** 2, axis=-1, keepdims=True)
    h0 = (h - mu) / jnp.sqrt(var + 1e-5) * ln_g_ref[...] + ln_b_ref[...]
    h0_ref[...] = h0
    s1_ref[...] = h0 @ gc1_w_ref[...]


def _layer_body(adj_ref, s_ref, hprev_ref, b_ref, wnext_ref,
                hnext_ref, snext_ref):
    out = adj_ref[...] @ s_ref[...] + b_ref[...]
    h = jnp.maximum(out, 0.0) + hprev_ref[...]
    hnext_ref[...] = h
    snext_ref[...] = h @ wnext_ref[...]


def _final_body(adj_ref, s_ref, b_ref, out_ref):
    out_ref[...] = adj_ref[...] @ s_ref[...] + b_ref[...]


def _row_strip(br):
    return pl.BlockSpec((br, _H), lambda i: (i, 0))


def _whole(shape):
    return pl.BlockSpec(shape, lambda i: (0,) * len(shape))


@functools.partial(jax.jit, static_argnums=())
def _gcn_stack(x, adj, mlp_w, mlp_b, ln_g, ln_b, gc1_w, gc1_b, gc2_w,
               gc2_b, gc3_w, gc3_b):
    f32 = jnp.float32
    h0, s1 = pl.pallas_call(
        _prologue_body,
        grid=(_N // _BP,),
        in_specs=[_row_strip(_BP), _whole((_H, _H)), _whole((1, _H)),
                  _whole((1, _H)), _whole((1, _H)), _whole((_H, _H))],
        out_specs=[_row_strip(_BP), _row_strip(_BP)],
        out_shape=[jax.ShapeDtypeStruct((_N, _H), f32),
                   jax.ShapeDtypeStruct((_N, _H), f32)],
    )(x, mlp_w, mlp_b.reshape(1, _H), ln_g.reshape(1, _H),
      ln_b.reshape(1, _H), gc1_w)

    layer = pl.pallas_call(
        _layer_body,
        grid=(_N // _BR,),
        in_specs=[pl.BlockSpec((_BR, _N), lambda i: (i, 0)),
                  _whole((_N, _H)), _row_strip(_BR), _whole((1, _H)),
                  _whole((_H, _H))],
        out_specs=[_row_strip(_BR), _row_strip(_BR)],
        out_shape=[jax.ShapeDtypeStruct((_N, _H), f32),
                   jax.ShapeDtypeStruct((_N, _H), f32)],
    )
    h1, s2 = layer(adj, s1, h0, gc1_b.reshape(1, _H), gc2_w)
    h2, hg = layer(adj, s2, h1, gc2_b.reshape(1, _H), gc3_w)

    x_hat = pl.pallas_call(
        _final_body,
        grid=(_N // _BR,),
        in_specs=[pl.BlockSpec((_BR, _N), lambda i: (i, 0)),
                  _whole((_N, _H)), _whole((1, _H))],
        out_specs=_row_strip(_BR),
        out_shape=jax.ShapeDtypeStruct((_N, _H), f32),
    )(adj, hg, gc3_b.reshape(1, _H))
    return x_hat, hg


def kernel(x, adj, attrs, mlp_w, mlp_b, ln_g, ln_b,
           gc1_w, gc1_b, gc2_w, gc2_b, gc3_w, gc3_b):
    x_hat, hg = _gcn_stack(x, adj, mlp_w, mlp_b, ln_g, ln_b, gc1_w, gc1_b,
                           gc2_w, gc2_b, gc3_w, gc3_b)

    cdf = jnp.cumsum(adj, axis=1)
    totals = cdf[:, -1:]
    u = jax.random.uniform(jax.random.key(42), (_N, _K),
                           dtype=jnp.float32) * totals
    samp = jax.vmap(lambda c, uu: jnp.searchsorted(c, uu))(cdf, u)
    samp = jnp.minimum(samp, _N - 1)

    neigh_attrs = attrs[samp]
    neigh_recon = hg[samp]
    return (x_hat, neigh_recon, neigh_attrs)


# speedup vs baseline: 1.4120x; 1.3915x over previous
import functools

import jax
import jax.numpy as jnp
from jax import lax
from jax.experimental import pallas as pl
from jax.experimental.pallas import tpu as pltpu
from jax.experimental.pallas import tpu_sc as plsc

_N = 10000
_H = 128
_K = 10
_BR = 200
_BP = 1000

_NB = 79
_NBP = 80
_RPAD = 10240
_NRC = _RPAD // 1024

_NC = 2
_NW = 32
_RCH = 8
_NCHUNK = _N // _RCH
_KP = 16


def _prologue_body(x_ref, mlp_w_ref, mlp_b_ref, ln_g_ref, ln_b_ref,
                   gc1_w_ref, h0_ref, s1_ref):
    h = jnp.maximum(x_ref[...] @ mlp_w_ref[...].T + mlp_b_ref[...], 0.0)
    mu = jnp.mean(h, axis=-1, keepdims=True)
    var = jnp.mean((h - mu) ** 2, axis=-1, keepdims=True)
    h0 = (h - mu) / jnp.sqrt(var + 1e-5) * ln_g_ref[...] + ln_b_ref[...]
    h0_ref[...] = h0
    s1_ref[...] = h0 @ gc1_w_ref[...]


def _layer_body(adj_ref, s_ref, hprev_ref, b_ref, wnext_ref,
                hnext_ref, snext_ref):
    out = adj_ref[...] @ s_ref[...] + b_ref[...]
    h = jnp.maximum(out, 0.0) + hprev_ref[...]
    hnext_ref[...] = h
    snext_ref[...] = h @ wnext_ref[...]


def _final_body(adj_ref, s_ref, b_ref, out_ref):
    out_ref[...] = adj_ref[...] @ s_ref[...] + b_ref[...]


def _fold_body(w_ref, out_ref, carry_ref):
    @pl.when(pl.program_id(1) == 0)
    def _():
        carry_ref[...] = jnp.zeros((8, 128), jnp.float32)

    p = w_ref[0]
    for j in range(1, 128):
        p = p + w_ref[j]
    c = carry_ref[...] + p
    carry_ref[...] = c
    out_ref[0] = c


def _row_strip(br):
    return pl.BlockSpec((br, _H), lambda i: (i, 0))


def _whole(shape):
    return pl.BlockSpec(shape, lambda i: (0,) * len(shape))


def _gcn_stack(x, adj, mlp_w, mlp_b, ln_g, ln_b, gc1_w, gc1_b, gc2_w,
               gc2_b, gc3_w, gc3_b):
    f32 = jnp.float32
    h0, s1 = pl.pallas_call(
        _prologue_body,
        grid=(_N // _BP,),
        in_specs=[_row_strip(_BP), _whole((_H, _H)), _whole((1, _H)),
                  _whole((1, _H)), _whole((1, _H)), _whole((_H, _H))],
        out_specs=[_row_strip(_BP), _row_strip(_BP)],
        out_shape=[jax.ShapeDtypeStruct((_N, _H), f32),
                   jax.ShapeDtypeStruct((_N, _H), f32)],
    )(x, mlp_w, mlp_b.reshape(1, _H), ln_g.reshape(1, _H),
      ln_b.reshape(1, _H), gc1_w)

    layer = pl.pallas_call(
        _layer_body,
        grid=(_N // _BR,),
        in_specs=[pl.BlockSpec((_BR, _N), lambda i: (i, 0)),
                  _whole((_N, _H)), _row_strip(_BR), _whole((1, _H)),
                  _whole((_H, _H))],
        out_specs=[_row_strip(_BR), _row_strip(_BR)],
        out_shape=[jax.ShapeDtypeStruct((_N, _H), f32),
                   jax.ShapeDtypeStruct((_N, _H), f32)],
    )
    h1, s2 = layer(adj, s1, h0, gc1_b.reshape(1, _H), gc2_w)
    h2, hg = layer(adj, s2, h1, gc2_b.reshape(1, _H), gc3_w)

    x_hat = pl.pallas_call(
        _final_body,
        grid=(_N // _BR,),
        in_specs=[pl.BlockSpec((_BR, _N), lambda i: (i, 0)),
                  _whole((_N, _H)), _whole((1, _H))],
        out_specs=_row_strip(_BR),
        out_shape=jax.ShapeDtypeStruct((_N, _H), f32),
    )(adj, hg, gc3_b.reshape(1, _H))
    return x_hat, hg


def _cdf_sub(adj):
    adjp = jnp.pad(adj, ((0, _RPAD - _N), (0, _NB * 128 - _N)))
    adjt3 = adjp.T.reshape(_NB * 128, _NBP, 128)
    folds = pl.pallas_call(
        _fold_body,
        grid=(_NRC, _NB),
        in_specs=[pl.BlockSpec((128, 8, 128), lambda rc, b: (b, rc, 0))],
        out_specs=pl.BlockSpec((1, 8, 128), lambda rc, b: (b, rc, 0)),
        out_shape=jax.ShapeDtypeStruct((_NB, _NBP, 128), jnp.float32),
        scratch_shapes=[pltpu.VMEM((8, 128), jnp.float32)],
    )(adjt3)
    return folds.transpose(1, 2, 0).reshape(_RPAD, _NB)[:_N]


def _sc_meshes():
    return plsc.VectorSubcoreMesh(core_axis_name="c", subcore_axis_name="s")


def _sample_kernel(cdfsub_pad, u_pad, adj16):
    @functools.partial(
        pl.kernel,
        out_type=jax.ShapeDtypeStruct((_N * _KP,), jnp.int32),
        mesh=_sc_meshes(),
        compiler_params=pltpu.CompilerParams(use_tc_tiling_on_sc=False, needs_layout_passes=False),
        scratch_types=[
            pltpu.VMEM((_RCH * 128,), jnp.float32),
            pltpu.VMEM((_RCH * _KP,), jnp.float32),
            pltpu.VMEM((_RCH * 128,), jnp.int32),
            pltpu.VMEM((_RCH, 128, 16), jnp.float32),
            pltpu.VMEM((_RCH * _KP,), jnp.int32),
            pltpu.SemaphoreType.DMA,
        ],
    )
    def k(cdfsub_hbm, u_hbm, adj16_hbm, idx_hbm,
          cdfs_v, u_v, winidx_v, win_v, idxout_v, sem):
        wid = lax.axis_index("s") * _NC + lax.axis_index("c")
        lanes = lax.iota(jnp.int32, 16)
        zf = jnp.zeros((16,), jnp.float32)
        zi = jnp.zeros((16,), jnp.int32)

        def chunk(it, _):
            ci = it * _NW + wid

            @pl.when(ci < _NCHUNK)
            def _():
                r0 = ci * _RCH
                pltpu.sync_copy(cdfsub_hbm.at[pl.ds(r0 * 128, _RCH * 128)],
                                cdfs_v)
                pltpu.sync_copy(u_hbm.at[pl.ds(r0 * _KP, _RCH * _KP)], u_v)
                nbs = []
                for row in range(_RCH):
                    u_row = u_v[pl.ds(row * _KP, _KP)]
                    pos = zi
                    for s in (64, 32, 16, 8, 4, 2, 1):
                        val = plsc.load_gather(
                            cdfs_v, [row * 128 + pos + (s - 1)])
                        pos = pos + jnp.where(val < u_row, s, 0)
                    nbs.append(pos)
                    w0 = jnp.minimum(pos * 128, _N - 128)
                    base16 = (r0 + row) * (_N // 16) + (w0 // 16)
                    for j8 in range(8):
                        plsc.store_scatter(
                            winidx_v, [row * 128 + lanes * 8 + j8],
                            base16 + j8)
                copies = [
                    pltpu.make_async_copy(
                        adj16_hbm.at[winidx_v.at[pl.ds(row * 128, 128)]],
                        win_v.at[row], sem)
                    for row in range(_RCH)
                ]
                for cp in copies:
                    cp.start()
                for cp in copies:
                    cp.wait()
                for row in range(_RCH):
                    rowv = jnp.full((16,), row, jnp.int32)
                    u_row = u_v[pl.ds(row * _KP, _KP)]
                    nb = nbs[row]
                    cm1 = plsc.load_gather(
                        cdfs_v, [row * 128 + jnp.maximum(nb - 1, 0)])
                    carry = jnp.where(nb > 0, cm1, 0.0)
                    j0 = nb * 128 - jnp.minimum(nb * 128, _N - 128)

                    def fine(j, st):
                        p, cnt = st
                        maj = lanes * 8 + (j // 16)
                        mnr = jnp.full((16,), 0, jnp.int32) + (j % 16)
                        wv = plsc.load_gather(win_v, [rowv, maj, mnr])
                        keep = j >= j0
                        p = p + jnp.where(keep, wv, 0.0)
                        cval = p + carry
                        cnt = cnt + jnp.where(keep & (cval < u_row), 1, 0)
                        return (p, cnt)

                    _, cnt = lax.fori_loop(0, 128, fine, (zf, zi), unroll=16)
                    idxf = jnp.minimum(nb * 128 + cnt, _N - 1)
                    idxout_v[pl.ds(row * _KP, _KP)] = idxf
                pltpu.sync_copy(idxout_v, idx_hbm.at[pl.ds(r0 * _KP,
                                                           _RCH * _KP)])

        lax.fori_loop(0, (_NCHUNK + _NW - 1) // _NW, chunk, None)

    return k(cdfsub_pad, u_pad, adj16)


def _gather_kernel(idx_flat, attrs, hg):
    @functools.partial(
        pl.kernel,
        out_type=[jax.ShapeDtypeStruct((_N * _KP, _H), jnp.float32),
                  jax.ShapeDtypeStruct((_N * _KP, _H), jnp.float32)],
        mesh=_sc_meshes(),
        compiler_params=pltpu.CompilerParams(use_tc_tiling_on_sc=False, needs_layout_passes=False),
        scratch_types=[
            pltpu.VMEM((_RCH * _KP,), jnp.int32),
            pltpu.VMEM((_RCH * _KP, _H), jnp.float32),
            pltpu.VMEM((_RCH * _KP, _H), jnp.float32),
            pltpu.SemaphoreType.DMA,
        ],
    )
    def k(idx_hbm, attrs_hbm, hg_hbm, outa_hbm, outh_hbm,
          idx_v, ga_v, gh_v, sem):
        wid = lax.axis_index("s") * _NC + lax.axis_index("c")

        def chunk(it, _):
            ci = it * _NW + wid

            @pl.when(ci < _NCHUNK)
            def _():
                base = ci * _RCH * _KP
                pltpu.sync_copy(idx_hbm.at[pl.ds(base, _RCH * _KP)], idx_v)
                cpa = pltpu.make_async_copy(attrs_hbm.at[idx_v], ga_v, sem)
                cph = pltpu.make_async_copy(hg_hbm.at[idx_v], gh_v, sem)
                cpa.start()
                cph.start()
                cpa.wait()
                cph.wait()
                pltpu.sync_copy(ga_v, outa_hbm.at[pl.ds(base, _RCH * _KP)])
                pltpu.sync_copy(gh_v, outh_hbm.at[pl.ds(base, _RCH * _KP)])

        lax.fori_loop(0, (_NCHUNK + _NW - 1) // _NW, chunk, None)

    return k(idx_flat, attrs, hg)


def kernel(x, adj, attrs, mlp_w, mlp_b, ln_g, ln_b,
           gc1_w, gc1_b, gc2_w, gc2_b, gc3_w, gc3_b):
    x_hat, hg = _gcn_stack(x, adj, mlp_w, mlp_b, ln_g, ln_b, gc1_w, gc1_b,
                           gc2_w, gc2_b, gc3_w, gc3_b)

    cdfsub = _cdf_sub(adj)
    totals = cdfsub[:, _NB - 1:]
    u = jax.random.uniform(jax.random.key(42), (_N, _K),
                           dtype=jnp.float32) * totals
    u_pad = jnp.pad(u, ((0, 0), (0, _KP - _K))).reshape(_N * _KP)
    cdfsub_pad = jnp.concatenate(
        [cdfsub, jnp.full((_N, 128 - _NB), jnp.inf, jnp.float32)],
        axis=1).reshape(_N * 128)
    adj16 = adj.reshape(_N * _N // 16, 16)

    idx_flat = _sample_kernel(cdfsub_pad, u_pad, adj16)
    outa, outh = _gather_kernel(idx_flat, attrs, hg)
    neigh_attrs = outa.reshape(_N, _KP, _H)[:, :_K, :]
    neigh_recon = outh.reshape(_N, _KP, _H)[:, :_K, :]
    return (x_hat, neigh_recon, neigh_attrs)
